# baseline (speedup 1.0000x reference)
import dataclasses
import functools

import jax
import jax.numpy as jnp
from jax import lax
from jax.experimental import pallas as pl
from jax.experimental.pallas import tpu as pltpu
from jax.experimental.pallas import tpu_sc as plsc

E = 8
T = 16384
D = 768
BM = 256
SCW = 128

@functools.cache
def _mesh():
    return plsc.VectorSubcoreMesh(core_axis_name="core", subcore_axis_name="subcore")


def _sc_params():
    cp = pltpu.CompilerParams()
    if "needs_layout_passes" in pltpu.CompilerParams.__dataclass_fields__:
        cp = dataclasses.replace(cp, needs_layout_passes=False)
    return cp


NW = 32
ROWS_W = T // NW
NCH = ROWS_W // SCW


def _worker_id():
    return lax.axis_index("core") * 16 + lax.axis_index("subcore")


NC = 2
HALF = T // NC
ROWS_WC = HALF // NW
NCH_C = ROWS_WC // SCW


def _dispatch_chunk(x, rw1, perm2c):
    @functools.partial(
        pl.kernel,
        out_type=(
            jax.ShapeDtypeStruct((HALF, D), jnp.float32),
            jax.ShapeDtypeStruct((HALF,), jnp.float32),
        ),
        mesh=_mesh(),
        compiler_params=_sc_params(),
        scratch_types=[
            pltpu.VMEM((ROWS_WC,), jnp.int32),
            pltpu.VMEM((SCW, D), jnp.float32),
            pltpu.VMEM((T,), jnp.float32),
            pltpu.VMEM((ROWS_WC,), jnp.float32),
        ],
    )
    def k(x_hbm, rw_hbm, perm_hbm, xs_hbm, rws_hbm, idx_v, buf_v, rwt_v, rwo_v):
        w = _worker_id()
        base = w * ROWS_WC
        pltpu.sync_copy(perm_hbm.at[w], idx_v)
        pltpu.sync_copy(rw_hbm, rwt_v)

        @pl.loop(0, ROWS_WC // 16)
        def _(kk):
            iv = idx_v[pl.ds(kk * 16, 16)]
            rwo_v[pl.ds(kk * 16, 16)] = plsc.load_gather(rwt_v, [iv])

        pltpu.sync_copy(rwo_v, rws_hbm.at[pl.ds(base, ROWS_WC)])

        @pl.loop(0, NCH_C)
        def _(j):
            pltpu.sync_copy(x_hbm.at[idx_v.at[pl.ds(j * SCW, SCW)]], buf_v)
            pltpu.sync_copy(buf_v, xs_hbm.at[pl.ds(base + j * SCW, SCW)])

    return k(x, rw1, perm2c)


def _undispatch(ys, perm3):
    @functools.partial(
        pl.kernel,
        out_type=jax.ShapeDtypeStruct((T, D), jnp.float32),
        mesh=_mesh(),
        scratch_types=[
            pltpu.VMEM((NCH, SCW), jnp.int32),
            pltpu.VMEM((SCW, D), jnp.float32),
        ],
    )
    def k(ys0_hbm, ys1_hbm, perm_hbm, o_hbm, idx_v, buf_v):
        w = _worker_id()
        pltpu.sync_copy(perm_hbm.at[w], idx_v)

        @pl.when(w < NW // 2)
        def _():
            base = w * ROWS_W

            @pl.loop(0, NCH)
            def _(j):
                pltpu.sync_copy(ys0_hbm.at[pl.ds(base + j * SCW, SCW)], buf_v)
                pltpu.sync_copy(buf_v, o_hbm.at[idx_v.at[j]])

        @pl.when(w >= NW // 2)
        def _():
            base = w * ROWS_W - HALF

            @pl.loop(0, NCH)
            def _(j):
                pltpu.sync_copy(ys1_hbm.at[pl.ds(base + j * SCW, SCW)], buf_v)
                pltpu.sync_copy(buf_v, o_hbm.at[idx_v.at[j]])

    return k(ys[0], ys[1], perm3)


NB = T // BM
NI = NB + E - 1


def _gmm_body(m_ref, g_ref, fv_ref, s_ref, t_ref, xs_ref, rws_ref, w_ref, b_ref, o_ref):
    i = pl.program_id(0)
    g = g_ref[i]
    row0 = m_ref[i] * BM
    rows = row0 + lax.broadcasted_iota(jnp.int32, (BM, 1), 0)
    mask = jnp.logical_and(rows >= s_ref[i], rows < t_ref[i])
    xm = jnp.where(mask, xs_ref[...], 0.0)
    w = w_ref[pl.ds(g, 1)][0]
    contrib = lax.dot_general(
        xm, w, (((1,), (1,)), ((), ())),
        preferred_element_type=jnp.float32,
    )
    bias = b_ref[pl.ds(g, 1)]
    contrib += jnp.where(mask, jnp.reshape(bias, (1, D)), 0.0)
    contrib *= rws_ref[...]

    @pl.when(fv_ref[i] == 1)
    def _():
        o_ref[...] = jnp.zeros((BM, D), jnp.float32)

    o_ref[...] += contrib


def _gmm(items, xs, rws, W, b):
    m_arr, g_arr, fv_arr, s_arr, t_arr = items
    grid_spec = pltpu.PrefetchScalarGridSpec(
        num_scalar_prefetch=5,
        grid=(m_arr.shape[0],),
        in_specs=[
            pl.BlockSpec((BM, D), lambda i, m, g, fv, s, t: (m[i], 0)),
            pl.BlockSpec((BM, 1), lambda i, m, g, fv, s, t: (m[i], 0)),
            pl.BlockSpec((E, D, D), lambda i, m, g, fv, s, t: (0, 0, 0)),
            pl.BlockSpec((E, D), lambda i, m, g, fv, s, t: (0, 0)),
        ],
        out_specs=pl.BlockSpec((BM, D), lambda i, m, g, fv, s, t: (m[i], 0)),
    )
    return pl.pallas_call(
        _gmm_body,
        grid_spec=grid_spec,
        out_shape=jax.ShapeDtypeStruct((xs.shape[0], D), jnp.float32),
    )(m_arr, g_arr, fv_arr, s_arr, t_arr, xs, rws, W, b)


NB_C = HALF // BM
NI_C = NB_C + E - 1


def _work_items(off, base):
    m = jnp.arange(NB_C, dtype=jnp.int32)[:, None]
    lo = jnp.maximum(off[:-1][None, :], base + m * BM)
    hi = jnp.minimum(off[1:][None, :], base + (m + 1) * BM)
    ov = (hi > lo).ravel()
    idx = jnp.nonzero(ov, size=NI_C, fill_value=0)[0].astype(jnp.int32)
    valid = jnp.arange(NI_C) < jnp.sum(ov.astype(jnp.int32))
    m_arr = jnp.where(valid, idx // E, NB_C - 1).astype(jnp.int32)
    g_arr = jnp.where(valid, idx % E, E - 1).astype(jnp.int32)
    s_arr = jnp.where(valid, lo.ravel()[idx] - base, 0).astype(jnp.int32)
    t_arr = jnp.where(valid, hi.ravel()[idx] - base, 0).astype(jnp.int32)
    fv_arr = jnp.concatenate(
        [jnp.ones((1,), jnp.int32), (m_arr[1:] != m_arr[:-1]).astype(jnp.int32)]
    )
    return m_arr, g_arr, fv_arr, s_arr, t_arr


def _scale_body(y_ref, rw_ref, o_ref):
    o_ref[...] = y_ref[...] * rw_ref[...]


def _scale(y, rw):
    BS = 2048
    return pl.pallas_call(
        _scale_body,
        grid=(T // BS,),
        in_specs=[
            pl.BlockSpec((BS, D), lambda i: (i, 0)),
            pl.BlockSpec((BS, 1), lambda i: (i, 0)),
        ],
        out_specs=pl.BlockSpec((BS, D), lambda i: (i, 0)),
        out_shape=jax.ShapeDtypeStruct((T, D), jnp.float32),
    )(y, rw)


def _routing_plan(selected_experts):
    sel = selected_experts[:, 0]
    perm = jnp.argsort(sel).astype(jnp.int32)
    counts = jnp.sum(
        (sel[:, None] == jnp.arange(E, dtype=jnp.int32)[None, :]).astype(jnp.int32),
        axis=0,
    )
    off = jnp.concatenate(
        [jnp.zeros((1,), jnp.int32), jnp.cumsum(counts).astype(jnp.int32)]
    )
    return perm, off


def kernel(hidden_states, selected_experts, routing_weights, W, b):
    perm, off = _routing_plan(selected_experts)
    rw1 = routing_weights[:, 0]
    permc = perm.reshape(NC, NW, ROWS_WC)
    ys = []
    for c in range(NC):
        xs_c, rws_c = _dispatch_chunk(hidden_states, rw1, permc[c])
        items_c = _work_items(off, c * HALF)
        ys.append(_gmm(items_c, xs_c, rws_c.reshape(HALF, 1), W, b))
    return _undispatch(ys, perm.reshape(NW, NCH, SCW))


# speedup vs baseline: 1.2408x; 1.2408x over previous
import dataclasses
import functools

import jax
import jax.numpy as jnp
from jax import lax
from jax.experimental import pallas as pl
from jax.experimental.pallas import tpu as pltpu
from jax.experimental.pallas import tpu_sc as plsc

E = 8
T = 16384
D = 768
BM = 512
SCW = 128

@functools.cache
def _mesh():
    return plsc.VectorSubcoreMesh(core_axis_name="core", subcore_axis_name="subcore")


def _sc_params():
    cp = pltpu.CompilerParams()
    if "needs_layout_passes" in pltpu.CompilerParams.__dataclass_fields__:
        cp = dataclasses.replace(cp, needs_layout_passes=False)
    return cp


NW = 32
ROWS_W = T // NW
NCH = ROWS_W // SCW


def _worker_id():
    return lax.axis_index("core") * 16 + lax.axis_index("subcore")


def _dispatch(x, rw1, perm2):
    @functools.partial(
        pl.kernel,
        out_type=(
            jax.ShapeDtypeStruct((T, D), jnp.float32),
            jax.ShapeDtypeStruct((T,), jnp.float32),
        ),
        mesh=_mesh(),
        compiler_params=_sc_params(),
        scratch_types=[
            pltpu.VMEM((ROWS_W,), jnp.int32),
            pltpu.VMEM((SCW, D), jnp.float32),
            pltpu.VMEM((T,), jnp.float32),
            pltpu.VMEM((ROWS_W,), jnp.float32),
        ],
    )
    def k(x_hbm, rw_hbm, perm_hbm, xs_hbm, rws_hbm, idx_v, buf_v, rwt_v, rwo_v):
        w = _worker_id()
        base = w * ROWS_W
        pltpu.sync_copy(perm_hbm.at[w], idx_v)
        pltpu.sync_copy(rw_hbm, rwt_v)

        @pl.loop(0, ROWS_W // 16)
        def _(kk):
            iv = idx_v[pl.ds(kk * 16, 16)]
            rwo_v[pl.ds(kk * 16, 16)] = plsc.load_gather(rwt_v, [iv])

        pltpu.sync_copy(rwo_v, rws_hbm.at[pl.ds(base, ROWS_W)])

        @pl.loop(0, NCH)
        def _(j):
            pltpu.sync_copy(x_hbm.at[idx_v.at[pl.ds(j * SCW, SCW)]], buf_v)
            pltpu.sync_copy(buf_v, xs_hbm.at[pl.ds(base + j * SCW, SCW)])

    return k(x, rw1, perm2)


def _undispatch(ys, perm3):
    @functools.partial(
        pl.kernel,
        out_type=jax.ShapeDtypeStruct((T, D), jnp.float32),
        mesh=_mesh(),
        scratch_types=[
            pltpu.VMEM((NCH, SCW), jnp.int32),
            pltpu.VMEM((SCW, D), jnp.float32),
        ],
    )
    def k(ys_hbm, perm_hbm, o_hbm, idx_v, buf_v):
        w = _worker_id()
        base = w * ROWS_W
        pltpu.sync_copy(perm_hbm.at[w], idx_v)

        @pl.loop(0, NCH)
        def _(j):
            r0 = base + j * SCW
            pltpu.sync_copy(ys_hbm.at[pl.ds(r0, SCW)], buf_v)
            pltpu.sync_copy(buf_v, o_hbm.at[idx_v.at[j]])

    return k(ys, perm3)


NB = T // BM
NI = NB + E - 1


def _gmm_body(m_ref, g_ref, fv_ref, s_ref, t_ref, xs_ref, rws_ref, w_ref, b_ref, o_ref):
    i = pl.program_id(0)
    g = g_ref[i]
    row0 = m_ref[i] * BM
    rows = row0 + lax.broadcasted_iota(jnp.int32, (BM, 1), 0)
    mask = jnp.logical_and(rows >= s_ref[i], rows < t_ref[i])
    xm = jnp.where(mask, xs_ref[...], 0.0)
    w = w_ref[pl.ds(g, 1)][0]
    contrib = lax.dot_general(
        xm, w, (((1,), (1,)), ((), ())),
        preferred_element_type=jnp.float32,
    )
    bias = b_ref[pl.ds(g, 1)]
    contrib += jnp.where(mask, jnp.reshape(bias, (1, D)), 0.0)
    contrib *= rws_ref[...]

    @pl.when(fv_ref[i] == 1)
    def _():
        o_ref[...] = jnp.zeros((BM, D), jnp.float32)

    o_ref[...] += contrib


def _gmm(items, xs, rws, W, b):
    m_arr, g_arr, fv_arr, s_arr, t_arr = items
    grid_spec = pltpu.PrefetchScalarGridSpec(
        num_scalar_prefetch=5,
        grid=(NI,),
        in_specs=[
            pl.BlockSpec((BM, D), lambda i, m, g, fv, s, t: (m[i], 0)),
            pl.BlockSpec((BM, 1), lambda i, m, g, fv, s, t: (m[i], 0)),
            pl.BlockSpec((E, D, D), lambda i, m, g, fv, s, t: (0, 0, 0)),
            pl.BlockSpec((E, D), lambda i, m, g, fv, s, t: (0, 0)),
        ],
        out_specs=pl.BlockSpec((BM, D), lambda i, m, g, fv, s, t: (m[i], 0)),
    )
    return pl.pallas_call(
        _gmm_body,
        grid_spec=grid_spec,
        out_shape=jax.ShapeDtypeStruct((T, D), jnp.float32),
    )(m_arr, g_arr, fv_arr, s_arr, t_arr, xs, rws, W, b)


def _work_items(off):
    m = jnp.arange(NB, dtype=jnp.int32)[:, None]
    e = jnp.arange(E, dtype=jnp.int32)[None, :]
    lo = jnp.maximum(off[:-1][None, :], m * BM)
    hi = jnp.minimum(off[1:][None, :], (m + 1) * BM)
    ov = (hi > lo).ravel()
    idx = jnp.nonzero(ov, size=NI, fill_value=0)[0].astype(jnp.int32)
    valid = jnp.arange(NI) < jnp.sum(ov.astype(jnp.int32))
    m_arr = jnp.where(valid, idx // E, NB - 1).astype(jnp.int32)
    g_arr = jnp.where(valid, idx % E, E - 1).astype(jnp.int32)
    s_arr = jnp.where(valid, lo.ravel()[idx], 0).astype(jnp.int32)
    t_arr = jnp.where(valid, hi.ravel()[idx], 0).astype(jnp.int32)
    fv_arr = jnp.concatenate(
        [jnp.ones((1,), jnp.int32), (m_arr[1:] != m_arr[:-1]).astype(jnp.int32)]
    )
    return m_arr, g_arr, fv_arr, s_arr, t_arr


def _scale_body(y_ref, rw_ref, o_ref):
    o_ref[...] = y_ref[...] * rw_ref[...]


def _scale(y, rw):
    BS = 2048
    return pl.pallas_call(
        _scale_body,
        grid=(T // BS,),
        in_specs=[
            pl.BlockSpec((BS, D), lambda i: (i, 0)),
            pl.BlockSpec((BS, 1), lambda i: (i, 0)),
        ],
        out_specs=pl.BlockSpec((BS, D), lambda i: (i, 0)),
        out_shape=jax.ShapeDtypeStruct((T, D), jnp.float32),
    )(y, rw)


def _routing_plan(selected_experts):
    sel = selected_experts[:, 0]
    perm = jnp.argsort(sel).astype(jnp.int32)
    counts = jnp.sum(
        (sel[:, None] == jnp.arange(E, dtype=jnp.int32)[None, :]).astype(jnp.int32),
        axis=0,
    )
    off = jnp.concatenate(
        [jnp.zeros((1,), jnp.int32), jnp.cumsum(counts).astype(jnp.int32)]
    )
    return perm, off


def kernel(hidden_states, selected_experts, routing_weights, W, b):
    perm, off = _routing_plan(selected_experts)
    items = _work_items(off)
    xs, rws = _dispatch(
        hidden_states, routing_weights[:, 0], perm.reshape(NW, ROWS_W)
    )
    ys = _gmm(items, xs, rws.reshape(T, 1), W, b)
    return _undispatch(ys, perm.reshape(NW, NCH, SCW))
